# baseline (speedup 1.0000x reference)
import functools

import jax
import jax.numpy as jnp
from jax import lax
from jax.experimental import pallas as pl
from jax.experimental.pallas import tpu as pltpu
from jax.experimental.pallas import tpu_sc as plsc

_INFO = plsc.get_sparse_core_info()
_NC = _INFO.num_cores
_NS = _INFO.num_subcores
_NW = _NC * _NS
_LANES = _INFO.num_lanes

_CHUNK = 320
_NBUF = 2


def _norm_table_body(tab_ref, gamma_ref, beta_ref, out_ref):
    x = tab_ref[...]
    mean = jnp.mean(x, axis=-1, keepdims=True)
    var = jnp.mean((x - mean) * (x - mean), axis=-1, keepdims=True)
    inv = lax.rsqrt(var + 1e-12)
    out_ref[...] = (x - mean) * inv * gamma_ref[...] + beta_ref[...]


def _norm_table(snp_table, ln_gamma, ln_beta):
    v, d = snp_table.shape
    return pl.pallas_call(
        _norm_table_body,
        out_shape=jax.ShapeDtypeStruct((v, d), jnp.float32),
    )(snp_table, ln_gamma.reshape(1, d), ln_beta.reshape(1, d))


_TC_BLK = 256


def _tc_expand_body(idx_ref, ntab_ref, out_ref):
    idx = idx_ref[...]
    n_rows, d = ntab_ref.shape
    acc = jnp.zeros(out_ref.shape, jnp.float32)
    for v in range(n_rows):
        row = ntab_ref[v, :].reshape(1, d)
        acc = acc + jnp.where(idx == v, 1.0, 0.0) * row
    out_ref[...] = acc


def _tc_expand(idx_tc, ntab):
    n_tc = idx_tc.shape[0]
    v, d = ntab.shape
    grid = (n_tc // _TC_BLK,)
    return pl.pallas_call(
        _tc_expand_body,
        grid=grid,
        in_specs=[
            pl.BlockSpec((_TC_BLK, 1), lambda i: (i, 0)),
            pl.BlockSpec((v, d), lambda i: (0, 0)),
        ],
        out_specs=pl.BlockSpec((_TC_BLK, d), lambda i: (i, 0)),
        out_shape=jax.ShapeDtypeStruct((n_tc, d), jnp.float32),
    )(idx_tc, ntab)


def _make_expand(n_tokens, n_rows, d):
    assert n_tokens % (_NW * _CHUNK) == 0
    per_w = n_tokens // _NW
    n_chunks = per_w // _CHUNK
    n_col = d // _LANES
    mesh = plsc.VectorSubcoreMesh(core_axis_name="c", subcore_axis_name="s")

    @functools.partial(
        pl.kernel,
        out_type=jax.ShapeDtypeStruct((n_tokens, d), jnp.float32),
        mesh=mesh,
        compiler_params=pltpu.CompilerParams(needs_layout_passes=False),
        scratch_types=[
            pltpu.VMEM_SHARED((_NS * per_w,), jnp.int32),
            pltpu.VMEM((n_rows, d), jnp.float32),
            pltpu.VMEM((_NBUF, _CHUNK, d), jnp.float32),
            pltpu.SMEM((_NBUF * _CHUNK,), jnp.int32),
            pltpu.SemaphoreType.DMA,
            pltpu.SemaphoreType.DMA,
            pltpu.SemaphoreType.DMA,
            pltpu.SemaphoreType.DMA,
        ],
    )
    def expand_kernel(idx_hbm, tab_hbm, out_hbm, idx_v, tab_v, rows_v, idx_sm,
                      semi0, semi1, sem0, sem1):
        sid = lax.axis_index("s")
        wid = sid * _NC + lax.axis_index("c")
        pltpu.sync_copy(idx_hbm.at[wid], idx_v.at[pl.ds(sid * per_w, per_w)])
        pltpu.sync_copy(tab_hbm, tab_v)
        base = wid * per_w
        sems = [sem0, sem1]
        semis = [semi0, semi1]

        def fire_idx(k):
            pltpu.async_copy(
                idx_v.at[pl.ds(sid * per_w + k * _CHUNK, _CHUNK)],
                idx_sm.at[pl.ds((k % _NBUF) * _CHUNK, _CHUNK)],
                semis[k % _NBUF],
            )

        def wait_idx(k):
            pltpu.make_async_copy(
                idx_v.at[pl.ds(0, _CHUNK)],
                idx_sm.at[pl.ds((k % _NBUF) * _CHUNK, _CHUNK)],
                semis[k % _NBUF],
            ).wait()

        def build(k, buf):
            @plsc.parallel_loop(0, _CHUNK, unroll=8)
            def _(t):
                v = idx_sm[buf * _CHUNK + t]
                for c in range(n_col):
                    sl = pl.ds(c * _LANES, _LANES)
                    rows_v[buf, t, sl] = tab_v[v, sl]

        def store(k, buf):
            pltpu.async_copy(
                rows_v.at[buf],
                out_hbm.at[pl.ds(base + k * _CHUNK, _CHUNK)],
                sems[buf],
            )

        def drain_store(buf):
            pltpu.make_async_copy(
                rows_v.at[buf],
                out_hbm.at[pl.ds(0, _CHUNK)],
                sems[buf],
            ).wait()

        fire_idx(0)
        for k in range(n_chunks):
            buf = k % _NBUF
            wait_idx(k)
            if k + 1 < n_chunks:
                fire_idx(k + 1)
            if k >= _NBUF:
                drain_store(buf)
            build(k, buf)
            store(k, buf)
        for buf in range(_NBUF):
            drain_store(buf)

    return expand_kernel


def kernel(snp, is_padding, snp_table, ln_gamma, ln_beta):
    b, l = snp.shape
    v, d = snp_table.shape
    n = b * l
    n_tc = 81920
    n_sc = n - n_tc
    ntab = _norm_table(snp_table, ln_gamma, ln_beta)
    flat = snp.reshape(n).astype(jnp.int32)
    idx_tc = flat[:n_tc].reshape(n_tc, 1)
    idx_sc = flat[n_tc:].reshape(_NW, n_sc // _NW)
    sc_out = _make_expand(n_sc, v, d)(idx_sc, ntab)
    tc_out = _tc_expand(idx_tc, ntab)
    out = jnp.concatenate([tc_out, sc_out], axis=0)
    return out.reshape(b, l, d), is_padding


# speedup vs baseline: 4.9615x; 4.9615x over previous
import functools

import jax
import jax.numpy as jnp
from jax import lax
from jax.experimental import pallas as pl
from jax.experimental.pallas import tpu as pltpu
from jax.experimental.pallas import tpu_sc as plsc

_INFO = plsc.get_sparse_core_info()
_NC = _INFO.num_cores
_NS = _INFO.num_subcores
_NW = _NC * _NS
_LANES = _INFO.num_lanes

_CHUNK = 400
_NBUF = 2


def _norm_table_body(tab_ref, gamma_ref, beta_ref, out_ref):
    x = tab_ref[...]
    mean = jnp.mean(x, axis=-1, keepdims=True)
    var = jnp.mean((x - mean) * (x - mean), axis=-1, keepdims=True)
    inv = lax.rsqrt(var + 1e-12)
    out_ref[...] = (x - mean) * inv * gamma_ref[...] + beta_ref[...]


def _norm_table(snp_table, ln_gamma, ln_beta):
    v, d = snp_table.shape
    return pl.pallas_call(
        _norm_table_body,
        out_shape=jax.ShapeDtypeStruct((v, d), jnp.float32),
    )(snp_table, ln_gamma.reshape(1, d), ln_beta.reshape(1, d))


def _make_expand(n_tokens, n_rows, d):
    assert n_tokens % (_NW * _CHUNK) == 0
    per_w = n_tokens // _NW
    n_chunks = per_w // _CHUNK
    n_col = d // _LANES
    mesh = plsc.VectorSubcoreMesh(core_axis_name="c", subcore_axis_name="s")

    @functools.partial(
        pl.kernel,
        out_type=jax.ShapeDtypeStruct((n_tokens, d), jnp.float32),
        mesh=mesh,
        compiler_params=pltpu.CompilerParams(needs_layout_passes=False),
        scratch_types=[
            pltpu.VMEM_SHARED((_NS * per_w,), jnp.int32),
            pltpu.VMEM((n_rows, d), jnp.float32),
            pltpu.VMEM((_NBUF, _CHUNK, d), jnp.float32),
            pltpu.SMEM((_NBUF * _CHUNK,), jnp.int32),
            pltpu.SemaphoreType.DMA,
            pltpu.SemaphoreType.DMA,
            pltpu.SemaphoreType.DMA,
            pltpu.SemaphoreType.DMA,
            pltpu.SemaphoreType.DMA,
        ],
    )
    def expand_kernel(idx_hbm, tab_hbm, out_hbm, idx_v, tab_v, rows_v, idx_sm,
                      semx, semi0, semi1, sem0, sem1):
        sid = lax.axis_index("s")
        wid = sid * _NC + lax.axis_index("c")
        pltpu.async_copy(
            idx_hbm.at[wid], idx_v.at[pl.ds(sid * per_w, per_w)], semx
        )
        pltpu.sync_copy(tab_hbm, tab_v)
        pltpu.make_async_copy(
            idx_hbm.at[0], idx_v.at[pl.ds(0, per_w)], semx
        ).wait()
        base = wid * per_w
        sems = [sem0, sem1]
        semis = [semi0, semi1]

        def fire_idx(k):
            pltpu.async_copy(
                idx_v.at[pl.ds(sid * per_w + k * _CHUNK, _CHUNK)],
                idx_sm.at[pl.ds((k % _NBUF) * _CHUNK, _CHUNK)],
                semis[k % _NBUF],
            )

        def wait_idx(k):
            pltpu.make_async_copy(
                idx_v.at[pl.ds(0, _CHUNK)],
                idx_sm.at[pl.ds((k % _NBUF) * _CHUNK, _CHUNK)],
                semis[k % _NBUF],
            ).wait()

        def build(k, buf):
            @plsc.parallel_loop(0, _CHUNK, unroll=8)
            def _(t):
                v = idx_sm[buf * _CHUNK + t]
                for c in range(n_col):
                    sl = pl.ds(c * _LANES, _LANES)
                    rows_v[buf, t, sl] = tab_v[v, sl]

        def store(k, buf):
            pltpu.async_copy(
                rows_v.at[buf],
                out_hbm.at[pl.ds(base + k * _CHUNK, _CHUNK)],
                sems[buf],
            )

        def drain_store(buf):
            pltpu.make_async_copy(
                rows_v.at[buf],
                out_hbm.at[pl.ds(0, _CHUNK)],
                sems[buf],
            ).wait()

        fire_idx(0)
        for k in range(n_chunks):
            buf = k % _NBUF
            wait_idx(k)
            if k + 1 < n_chunks:
                fire_idx(k + 1)
            if k >= _NBUF:
                drain_store(buf)
            build(k, buf)
            store(k, buf)
        for buf in range(_NBUF):
            drain_store(buf)

    return expand_kernel


def kernel(snp, is_padding, snp_table, ln_gamma, ln_beta):
    b, l = snp.shape
    v, d = snp_table.shape
    n = b * l
    ntab = _norm_table(snp_table, ln_gamma, ln_beta)
    idx = snp.reshape(_NW, n // _NW).astype(jnp.int32)
    out = _make_expand(n, v, d)(idx, ntab)
    return out.reshape(b, l, d), is_padding
